# baseline (speedup 1.0000x reference)
import functools

import jax
import jax.numpy as jnp
from jax import lax
from jax.experimental import pallas as pl
from jax.experimental.pallas import tpu as pltpu
from jax.experimental.pallas import tpu_sc as plsc

EMB = 128

_NC = 2
_NS = 16
_NW = _NC * _NS
_CHUNK = 64
_GRP = 32
_NGRP0 = 7
_NGRP1 = 3
_CHUNKS0 = _GRP * _NGRP0
_CHUNKS1 = _GRP * _NGRP1
_E_PAD = _NS * _CHUNK * (_CHUNKS0 + _CHUNKS1)
_ACC_ROWS = 10240


def _seg_scatter_body(table, src, dst, ef, out,
                      src_g, dst_g, rows0, rows1, ef0, ef1, acc,
                      sem_r0, sem_r1, sem_e0, sem_e1, sem_s0, sem_s1):
    cid = lax.axis_index("c")
    sid = lax.axis_index("s")
    ngrp = jnp.where(cid == 0, _NGRP0, _NGRP1)
    grow0 = jnp.where(cid == 0, sid * _CHUNKS0,
                      _NS * _CHUNKS0 + sid * _CHUNKS1)

    def zloop(r, carry):
        for d in range(EMB // 16):
            rows0[r, pl.ds(d * 16, 16)] = jnp.zeros((16,), jnp.float32)
        return carry
    lax.fori_loop(0, _CHUNK, zloop, 0)
    rows_per_tile = acc.shape[0] // _NS
    for k in range(rows_per_tile // _CHUNK):
        pltpu.sync_copy(rows0,
                        acc.at[pl.ds(sid * rows_per_tile + k * _CHUNK, _CHUNK)])
    plsc.subcore_barrier()

    def scale(rows_b, ef_b):
        def body(e, c2):
            s = ef_b[e, :]
            for d in range(EMB // 16):
                rows_b[e, pl.ds(d * 16, 16)] = rows_b[e, pl.ds(d * 16, 16)] * s
            return c2
        lax.fori_loop(0, _CHUNK, body, 0)

    def grp_body(grp, gcarry):
        gbase = grow0 + grp * _GRP
        pltpu.sync_copy(src.at[pl.ds(gbase, _GRP)], src_g)
        pltpu.sync_copy(dst.at[pl.ds(gbase, _GRP)], dst_g)
        pltpu.async_copy(table.at[src_g.at[0]], rows0, sem_r0)
        pltpu.async_copy(ef.at[pl.ds(gbase * _CHUNK, _CHUNK)], ef0, sem_e0)

        def pair(t, carry):
            j0 = 2 * t
            j1 = 2 * t + 1
            e0base = (gbase + j0) * _CHUNK
            pltpu.async_copy(table.at[src_g.at[j1]], rows1, sem_r1)
            pltpu.async_copy(ef.at[pl.ds(e0base + _CHUNK, _CHUNK)], ef1, sem_e1)
            pltpu.make_async_copy(table.at[src_g.at[j0]], rows0, sem_r0).wait()
            pltpu.make_async_copy(ef.at[pl.ds(e0base, _CHUNK)], ef0, sem_e0).wait()
            scale(rows0, ef0)
            pltpu.async_copy(rows0, acc.at[dst_g.at[j0]], sem_s0, add=True)

            pltpu.make_async_copy(table.at[src_g.at[j1]], rows1, sem_r1).wait()
            pltpu.make_async_copy(ef.at[pl.ds(e0base + _CHUNK, _CHUNK)],
                                  ef1, sem_e1).wait()

            @pl.when(t < _GRP // 2 - 1)
            def _():
                pltpu.make_async_copy(rows0, acc.at[dst_g.at[j0]], sem_s0).wait()
                pltpu.async_copy(table.at[src_g.at[j0 + 2]], rows0, sem_r0)
                pltpu.async_copy(ef.at[pl.ds(e0base + 2 * _CHUNK, _CHUNK)],
                                 ef0, sem_e0)

            @pl.when(t == _GRP // 2 - 1)
            def _():
                pltpu.make_async_copy(rows0, acc.at[dst_g.at[j0]], sem_s0).wait()

            scale(rows1, ef1)
            pltpu.async_copy(rows1, acc.at[dst_g.at[j1]], sem_s1, add=True)
            pltpu.make_async_copy(rows1, acc.at[dst_g.at[j1]], sem_s1).wait()
            return carry
        lax.fori_loop(0, _GRP // 2, pair, 0)
        return gcarry
    lax.fori_loop(0, ngrp, grp_body, 0)

    plsc.subcore_barrier()
    pltpu.sync_copy(acc.at[pl.ds(sid * rows_per_tile, rows_per_tile)],
                    out.at[cid, pl.ds(sid * rows_per_tile, rows_per_tile)])


def _seg_scatter(table, src, dst, ef):
    mesh = plsc.VectorSubcoreMesh(core_axis_name="c", subcore_axis_name="s",
                                  num_cores=_NC, num_subcores=_NS)
    return pl.kernel(
        _seg_scatter_body,
        out_type=jax.ShapeDtypeStruct((_NC, _ACC_ROWS, EMB), jnp.float32),
        mesh=mesh,
        scratch_types=[
            pltpu.VMEM((_GRP, _CHUNK), jnp.int32),
            pltpu.VMEM((_GRP, _CHUNK), jnp.int32),
            pltpu.VMEM((_CHUNK, EMB), jnp.float32),
            pltpu.VMEM((_CHUNK, EMB), jnp.float32),
            pltpu.VMEM((_CHUNK, 16), jnp.float32),
            pltpu.VMEM((_CHUNK, 16), jnp.float32),
            pltpu.VMEM_SHARED((_ACC_ROWS, EMB), jnp.float32),
            pltpu.SemaphoreType.DMA,
            pltpu.SemaphoreType.DMA,
            pltpu.SemaphoreType.DMA,
            pltpu.SemaphoreType.DMA,
            pltpu.SemaphoreType.DMA,
            pltpu.SemaphoreType.DMA,
        ],
    )(table, src, dst, ef)


def _mlp2_body(x_ref, w1_ref, b1_ref, w2_ref, b2_ref, o_ref):
    h = jnp.maximum(jnp.dot(x_ref[...], w1_ref[...],
                            preferred_element_type=jnp.float32) + b1_ref[...], 0.0)
    o_ref[...] = jnp.dot(h, w2_ref[...], preferred_element_type=jnp.float32) + b2_ref[...]


def _mlp2(x, w1, b1, w2, b2, block_rows):
    n, din = x.shape
    grid = n // block_rows
    return pl.pallas_call(
        _mlp2_body,
        grid=(grid,),
        in_specs=[
            pl.BlockSpec((block_rows, din), lambda i: (i, 0)),
            pl.BlockSpec((din, EMB), lambda i: (0, 0)),
            pl.BlockSpec((1, EMB), lambda i: (0, 0)),
            pl.BlockSpec((EMB, EMB), lambda i: (0, 0)),
            pl.BlockSpec((1, EMB), lambda i: (0, 0)),
        ],
        out_specs=pl.BlockSpec((block_rows, EMB), lambda i: (i, 0)),
        out_shape=jax.ShapeDtypeStruct((n, EMB), jnp.float32),
    )(x, w1, b1.reshape(1, EMB), w2, b2.reshape(1, EMB))


def _emb_body(x_ref, w_ref, b_ref, o_ref):
    o_ref[...] = jnp.maximum(
        jnp.dot(x_ref[...], w_ref[...], preferred_element_type=jnp.float32)
        + b_ref[...], 0.0)


def _emb(x, w, b, block_rows):
    n, din = x.shape
    grid = n // block_rows
    return pl.pallas_call(
        _emb_body,
        grid=(grid,),
        in_specs=[
            pl.BlockSpec((block_rows, din), lambda i: (i, 0)),
            pl.BlockSpec((din, EMB), lambda i: (0, 0)),
            pl.BlockSpec((1, EMB), lambda i: (0, 0)),
        ],
        out_specs=pl.BlockSpec((block_rows, EMB), lambda i: (i, 0)),
        out_shape=jax.ShapeDtypeStruct((n, EMB), jnp.float32),
    )(x, w, b.reshape(1, EMB))


def _omlp_body(part_ref, prev_ref, wa_ref, wb_ref, b1_ref, w2_ref, b2_ref, o_ref,
               *, outer_relu, n_conv_blocks):
    conv = part_ref[0] + part_ref[1]
    if n_conv_blocks is not None:
        live = (pl.program_id(0) < n_conv_blocks).astype(jnp.float32)
        conv = conv * live
    h = (jnp.dot(conv, wa_ref[...], preferred_element_type=jnp.float32)
         + jnp.dot(prev_ref[...], wb_ref[...], preferred_element_type=jnp.float32)
         + b1_ref[...])
    h = jnp.maximum(h, 0.0)
    o = jnp.dot(h, w2_ref[...], preferred_element_type=jnp.float32) + b2_ref[...]
    if outer_relu:
        o = jnp.maximum(o, 0.0)
    o_ref[...] = o


def _omlp(part, prev, w1, b1, w2, b2, block_rows, outer_relu, n_live):
    n = prev.shape[0]
    grid = n // block_rows
    nb = n_live // block_rows
    n_conv_blocks = None if n_live == n else nb
    wa = w1[:EMB]
    wb = w1[EMB:]
    body = functools.partial(_omlp_body, outer_relu=outer_relu,
                             n_conv_blocks=n_conv_blocks)
    return pl.pallas_call(
        body,
        grid=(grid,),
        in_specs=[
            pl.BlockSpec((2, block_rows, EMB),
                         lambda i: (0, jnp.minimum(i, nb - 1), 0)),
            pl.BlockSpec((block_rows, EMB), lambda i: (i, 0)),
            pl.BlockSpec((EMB, EMB), lambda i: (0, 0)),
            pl.BlockSpec((EMB, EMB), lambda i: (0, 0)),
            pl.BlockSpec((1, EMB), lambda i: (0, 0)),
            pl.BlockSpec((EMB, EMB), lambda i: (0, 0)),
            pl.BlockSpec((1, EMB), lambda i: (0, 0)),
        ],
        out_specs=pl.BlockSpec((block_rows, EMB), lambda i: (i, 0)),
        out_shape=jax.ShapeDtypeStruct((n, EMB), jnp.float32),
    )(part, prev, wa, wb, b1.reshape(1, EMB), w2, b2.reshape(1, EMB))


def _means_body(vf_ref, cf_ref, o_ref):
    vm = jnp.mean(vf_ref[...], axis=0, keepdims=True)
    cm = jnp.mean(cf_ref[...], axis=0, keepdims=True)
    o_ref[...] = jnp.concatenate([vm, cm], axis=1)[None]


def _readout_body(ff_ref, w1_ref, b1_ref, w2_ref, o_ref):
    h = jnp.maximum(jnp.dot(ff_ref[...], w1_ref[...],
                            preferred_element_type=jnp.float32) + b1_ref[...], 0.0)
    o_ref[...] = jnp.dot(h, w2_ref[...], preferred_element_type=jnp.float32)


def _bgc(p, left, src_idx, dst_idx, ef, right, out_size, right_to_left,
         outer_relu, n_active):
    fl = _mlp2(left, p['fl1_W'], p['fl1_b'], p['fl2_W'], p['fl2_b'],
               block_rows=2000)
    fr = _mlp2(right, p['fr1_W'], jnp.zeros((EMB,), jnp.float32),
               p['fr2_W'], p['fr2_b'], block_rows=2000)
    if right_to_left:
        prev, src_tab = fl, fr
    else:
        prev, src_tab = fr, fl
    part = _seg_scatter(src_tab, src_idx, dst_idx, ef)
    return _omlp(part, prev, p['o1_W'], p['o1_b'], p['o2_W'], p['o2_b'],
                 block_rows=2000, outer_relu=outer_relu, n_live=n_active)


def kernel(constraint_features, edge_indices, edge_features, variable_features,
           n_cons_total, n_vars_total, n_cons_small, n_vars_small, params):
    n_cons = constraint_features.shape[0]
    n_vars = variable_features.shape[0]
    n_cons_small_s = 1000
    n_vars_small_s = 2000
    p = params

    ei = edge_indices.astype(jnp.int32)
    n_edges = ei.shape[1]
    pad = _E_PAD - n_edges
    src_c = jnp.concatenate([ei[0], jnp.zeros((pad,), jnp.int32)]
                            ).reshape(_E_PAD // _CHUNK, _CHUNK)
    src_v = jnp.concatenate([ei[1], jnp.zeros((pad,), jnp.int32)]
                            ).reshape(_E_PAD // _CHUNK, _CHUNK)
    ef = jnp.concatenate([edge_features[:, 0], jnp.zeros((pad,), jnp.float32)])
    ef = jnp.asarray(jnp.broadcast_to(ef[:, None], (_E_PAD, 16)))

    cf = _emb(constraint_features, p['ce_W'], p['ce_b'], block_rows=2000)
    vf = _emb(variable_features, p['ve_W'], p['ve_b'], block_rows=2000)

    cf = _bgc(p['vtc'], cf, src_v, src_c, ef, vf, n_cons, True, True, n_cons)
    vf = _bgc(p['ctv'], cf, src_c, src_v, ef, vf, n_vars, False, True, n_cons)
    cf = _bgc(p['vtc2'], cf, src_v, src_c, ef, vf, n_cons, True, True, n_cons)
    vf = _bgc(p['ctv2'], cf, src_c, src_v, ef, vf, n_vars, False, True, n_cons)

    n_groups = n_vars // n_vars_small_s
    ff = pl.pallas_call(
        _means_body,
        grid=(n_groups,),
        in_specs=[
            pl.BlockSpec((n_vars_small_s, EMB), lambda i: (i, 0)),
            pl.BlockSpec((n_cons_small_s, EMB), lambda i: (i, 0)),
        ],
        out_specs=pl.BlockSpec((1, 1, 2 * EMB), lambda i: (i, 0, 0)),
        out_shape=jax.ShapeDtypeStruct((n_groups, 1, 2 * EMB), jnp.float32),
    )(vf, cf).reshape(n_groups, 2 * EMB)

    out = pl.pallas_call(
        _readout_body,
        in_specs=[
            pl.BlockSpec((n_groups, 2 * EMB), lambda i: (0, 0)),
            pl.BlockSpec((2 * EMB, EMB), lambda i: (0, 0)),
            pl.BlockSpec((1, EMB), lambda i: (0, 0)),
            pl.BlockSpec((EMB, 1), lambda i: (0, 0)),
        ],
        grid=(1,),
        out_specs=pl.BlockSpec((n_groups, 1), lambda i: (0, 0)),
        out_shape=jax.ShapeDtypeStruct((n_groups, 1), jnp.float32),
    )(ff, p['fo1_W'], p['fo1_b'].reshape(1, EMB), p['fo2_W'])
    return out


# speedup vs baseline: 1.1019x; 1.1019x over previous
import functools

import jax
import jax.numpy as jnp
from jax import lax
from jax.experimental import pallas as pl
from jax.experimental.pallas import tpu as pltpu
from jax.experimental.pallas import tpu_sc as plsc

EMB = 128

_NC = 2
_NS = 16
_NW = _NC * _NS
_CHUNK = 64
_GRP = 32
_NGRP0 = 9
_NGRP1 = 1
_CHUNKS0 = _GRP * _NGRP0
_CHUNKS1 = _GRP * _NGRP1
_E_PAD = _NS * _CHUNK * (_CHUNKS0 + _CHUNKS1)
_ACC_ROWS = 10240


def _seg_scatter_body(table, src, dst, ef, out,
                      src_g, dst_g, rows0, rows1, ef0, ef1, acc,
                      sem_r0, sem_r1, sem_e0, sem_e1, sem_s0, sem_s1):
    cid = lax.axis_index("c")
    sid = lax.axis_index("s")
    ngrp = jnp.where(cid == 0, _NGRP0, _NGRP1)
    grow0 = jnp.where(cid == 0, sid * _CHUNKS0,
                      _NS * _CHUNKS0 + sid * _CHUNKS1)

    def zloop(r, carry):
        for d in range(EMB // 16):
            rows0[r, pl.ds(d * 16, 16)] = jnp.zeros((16,), jnp.float32)
        return carry
    lax.fori_loop(0, _CHUNK, zloop, 0)
    rows_per_tile = acc.shape[0] // _NS
    for k in range(rows_per_tile // _CHUNK):
        pltpu.sync_copy(rows0,
                        acc.at[pl.ds(sid * rows_per_tile + k * _CHUNK, _CHUNK)])
    plsc.subcore_barrier()

    def scale(rows_b, ef_b):
        def body(e, c2):
            s = ef_b[e, :]
            for d in range(EMB // 16):
                rows_b[e, pl.ds(d * 16, 16)] = rows_b[e, pl.ds(d * 16, 16)] * s
            return c2
        lax.fori_loop(0, _CHUNK, body, 0)

    def grp_body(grp, gcarry):
        gbase = grow0 + grp * _GRP
        pltpu.sync_copy(src.at[pl.ds(gbase, _GRP)], src_g)
        pltpu.sync_copy(dst.at[pl.ds(gbase, _GRP)], dst_g)
        pltpu.async_copy(table.at[src_g.at[0]], rows0, sem_r0)
        pltpu.async_copy(ef.at[pl.ds(gbase * _CHUNK, _CHUNK)], ef0, sem_e0)

        def pair(t, carry):
            j0 = 2 * t
            j1 = 2 * t + 1
            e0base = (gbase + j0) * _CHUNK
            pltpu.async_copy(table.at[src_g.at[j1]], rows1, sem_r1)
            pltpu.async_copy(ef.at[pl.ds(e0base + _CHUNK, _CHUNK)], ef1, sem_e1)
            pltpu.make_async_copy(table.at[src_g.at[j0]], rows0, sem_r0).wait()
            pltpu.make_async_copy(ef.at[pl.ds(e0base, _CHUNK)], ef0, sem_e0).wait()
            scale(rows0, ef0)
            pltpu.async_copy(rows0, acc.at[dst_g.at[j0]], sem_s0, add=True)

            pltpu.make_async_copy(table.at[src_g.at[j1]], rows1, sem_r1).wait()
            pltpu.make_async_copy(ef.at[pl.ds(e0base + _CHUNK, _CHUNK)],
                                  ef1, sem_e1).wait()

            @pl.when(t < _GRP // 2 - 1)
            def _():
                pltpu.make_async_copy(rows0, acc.at[dst_g.at[j0]], sem_s0).wait()
                pltpu.async_copy(table.at[src_g.at[j0 + 2]], rows0, sem_r0)
                pltpu.async_copy(ef.at[pl.ds(e0base + 2 * _CHUNK, _CHUNK)],
                                 ef0, sem_e0)

            @pl.when(t == _GRP // 2 - 1)
            def _():
                pltpu.make_async_copy(rows0, acc.at[dst_g.at[j0]], sem_s0).wait()

            scale(rows1, ef1)
            pltpu.async_copy(rows1, acc.at[dst_g.at[j1]], sem_s1, add=True)
            pltpu.make_async_copy(rows1, acc.at[dst_g.at[j1]], sem_s1).wait()
            return carry
        lax.fori_loop(0, _GRP // 2, pair, 0)
        return gcarry
    lax.fori_loop(0, ngrp, grp_body, 0)

    plsc.subcore_barrier()
    pltpu.sync_copy(acc.at[pl.ds(sid * rows_per_tile, rows_per_tile)],
                    out.at[cid, pl.ds(sid * rows_per_tile, rows_per_tile)])


def _seg_scatter(table, src, dst, ef):
    mesh = plsc.VectorSubcoreMesh(core_axis_name="c", subcore_axis_name="s",
                                  num_cores=_NC, num_subcores=_NS)
    return pl.kernel(
        _seg_scatter_body,
        out_type=jax.ShapeDtypeStruct((_NC, _ACC_ROWS, EMB), jnp.float32),
        mesh=mesh,
        scratch_types=[
            pltpu.VMEM((_GRP, _CHUNK), jnp.int32),
            pltpu.VMEM((_GRP, _CHUNK), jnp.int32),
            pltpu.VMEM((_CHUNK, EMB), jnp.float32),
            pltpu.VMEM((_CHUNK, EMB), jnp.float32),
            pltpu.VMEM((_CHUNK, 16), jnp.float32),
            pltpu.VMEM((_CHUNK, 16), jnp.float32),
            pltpu.VMEM_SHARED((_ACC_ROWS, EMB), jnp.float32),
            pltpu.SemaphoreType.DMA,
            pltpu.SemaphoreType.DMA,
            pltpu.SemaphoreType.DMA,
            pltpu.SemaphoreType.DMA,
            pltpu.SemaphoreType.DMA,
            pltpu.SemaphoreType.DMA,
        ],
    )(table, src, dst, ef)


def _mlp2_body(x_ref, w1_ref, b1_ref, w2_ref, b2_ref, o_ref):
    h = jnp.maximum(jnp.dot(x_ref[...], w1_ref[...],
                            preferred_element_type=jnp.float32) + b1_ref[...], 0.0)
    o_ref[...] = jnp.dot(h, w2_ref[...], preferred_element_type=jnp.float32) + b2_ref[...]


def _mlp2(x, w1, b1, w2, b2, block_rows):
    n, din = x.shape
    grid = n // block_rows
    return pl.pallas_call(
        _mlp2_body,
        grid=(grid,),
        in_specs=[
            pl.BlockSpec((block_rows, din), lambda i: (i, 0)),
            pl.BlockSpec((din, EMB), lambda i: (0, 0)),
            pl.BlockSpec((1, EMB), lambda i: (0, 0)),
            pl.BlockSpec((EMB, EMB), lambda i: (0, 0)),
            pl.BlockSpec((1, EMB), lambda i: (0, 0)),
        ],
        out_specs=pl.BlockSpec((block_rows, EMB), lambda i: (i, 0)),
        out_shape=jax.ShapeDtypeStruct((n, EMB), jnp.float32),
    )(x, w1, b1.reshape(1, EMB), w2, b2.reshape(1, EMB))


def _emb_body(x_ref, w_ref, b_ref, o_ref):
    o_ref[...] = jnp.maximum(
        jnp.dot(x_ref[...], w_ref[...], preferred_element_type=jnp.float32)
        + b_ref[...], 0.0)


def _emb(x, w, b, block_rows):
    n, din = x.shape
    grid = n // block_rows
    return pl.pallas_call(
        _emb_body,
        grid=(grid,),
        in_specs=[
            pl.BlockSpec((block_rows, din), lambda i: (i, 0)),
            pl.BlockSpec((din, EMB), lambda i: (0, 0)),
            pl.BlockSpec((1, EMB), lambda i: (0, 0)),
        ],
        out_specs=pl.BlockSpec((block_rows, EMB), lambda i: (i, 0)),
        out_shape=jax.ShapeDtypeStruct((n, EMB), jnp.float32),
    )(x, w, b.reshape(1, EMB))


def _omlp_body(part_ref, prev_ref, wa_ref, wb_ref, b1_ref, w2_ref, b2_ref, o_ref,
               *, outer_relu, n_conv_blocks):
    conv = part_ref[0] + part_ref[1]
    if n_conv_blocks is not None:
        live = (pl.program_id(0) < n_conv_blocks).astype(jnp.float32)
        conv = conv * live
    h = (jnp.dot(conv, wa_ref[...], preferred_element_type=jnp.float32)
         + jnp.dot(prev_ref[...], wb_ref[...], preferred_element_type=jnp.float32)
         + b1_ref[...])
    h = jnp.maximum(h, 0.0)
    o = jnp.dot(h, w2_ref[...], preferred_element_type=jnp.float32) + b2_ref[...]
    if outer_relu:
        o = jnp.maximum(o, 0.0)
    o_ref[...] = o


def _omlp(part, prev, w1, b1, w2, b2, block_rows, outer_relu, n_live):
    n = prev.shape[0]
    grid = n // block_rows
    nb = n_live // block_rows
    n_conv_blocks = None if n_live == n else nb
    wa = w1[:EMB]
    wb = w1[EMB:]
    body = functools.partial(_omlp_body, outer_relu=outer_relu,
                             n_conv_blocks=n_conv_blocks)
    return pl.pallas_call(
        body,
        grid=(grid,),
        in_specs=[
            pl.BlockSpec((2, block_rows, EMB),
                         lambda i: (0, jnp.minimum(i, nb - 1), 0)),
            pl.BlockSpec((block_rows, EMB), lambda i: (i, 0)),
            pl.BlockSpec((EMB, EMB), lambda i: (0, 0)),
            pl.BlockSpec((EMB, EMB), lambda i: (0, 0)),
            pl.BlockSpec((1, EMB), lambda i: (0, 0)),
            pl.BlockSpec((EMB, EMB), lambda i: (0, 0)),
            pl.BlockSpec((1, EMB), lambda i: (0, 0)),
        ],
        out_specs=pl.BlockSpec((block_rows, EMB), lambda i: (i, 0)),
        out_shape=jax.ShapeDtypeStruct((n, EMB), jnp.float32),
    )(part, prev, wa, wb, b1.reshape(1, EMB), w2, b2.reshape(1, EMB))


def _means_body(vf_ref, cf_ref, o_ref):
    vm = jnp.mean(vf_ref[...], axis=0, keepdims=True)
    cm = jnp.mean(cf_ref[...], axis=0, keepdims=True)
    o_ref[...] = jnp.concatenate([vm, cm], axis=1)[None]


def _readout_body(ff_ref, w1_ref, b1_ref, w2_ref, o_ref):
    h = jnp.maximum(jnp.dot(ff_ref[...], w1_ref[...],
                            preferred_element_type=jnp.float32) + b1_ref[...], 0.0)
    o_ref[...] = jnp.dot(h, w2_ref[...], preferred_element_type=jnp.float32)


def _bgc(p, left, src_idx, dst_idx, ef, right, out_size, right_to_left,
         outer_relu, n_active):
    fl = _mlp2(left, p['fl1_W'], p['fl1_b'], p['fl2_W'], p['fl2_b'],
               block_rows=2000)
    fr = _mlp2(right, p['fr1_W'], jnp.zeros((EMB,), jnp.float32),
               p['fr2_W'], p['fr2_b'], block_rows=2000)
    if right_to_left:
        prev, src_tab = fl, fr
    else:
        prev, src_tab = fr, fl
    part = _seg_scatter(src_tab, src_idx, dst_idx, ef)
    return _omlp(part, prev, p['o1_W'], p['o1_b'], p['o2_W'], p['o2_b'],
                 block_rows=2000, outer_relu=outer_relu, n_live=n_active)


def kernel(constraint_features, edge_indices, edge_features, variable_features,
           n_cons_total, n_vars_total, n_cons_small, n_vars_small, params):
    n_cons = constraint_features.shape[0]
    n_vars = variable_features.shape[0]
    n_cons_small_s = 1000
    n_vars_small_s = 2000
    p = params

    ei = edge_indices.astype(jnp.int32)
    n_edges = ei.shape[1]
    pad = _E_PAD - n_edges
    src_c = jnp.concatenate([ei[0], jnp.zeros((pad,), jnp.int32)]
                            ).reshape(_E_PAD // _CHUNK, _CHUNK)
    src_v = jnp.concatenate([ei[1], jnp.zeros((pad,), jnp.int32)]
                            ).reshape(_E_PAD // _CHUNK, _CHUNK)
    ef = jnp.concatenate([edge_features[:, 0], jnp.zeros((pad,), jnp.float32)])
    ef = jnp.asarray(jnp.broadcast_to(ef[:, None], (_E_PAD, 16)))

    cf = _emb(constraint_features, p['ce_W'], p['ce_b'], block_rows=2000)
    vf = _emb(variable_features, p['ve_W'], p['ve_b'], block_rows=2000)

    cf = _bgc(p['vtc'], cf, src_v, src_c, ef, vf, n_cons, True, True, n_cons)
    vf = _bgc(p['ctv'], cf, src_c, src_v, ef, vf, n_vars, False, True, n_cons)
    cf = _bgc(p['vtc2'], cf, src_v, src_c, ef, vf, n_cons, True, True, n_cons)
    vf = _bgc(p['ctv2'], cf, src_c, src_v, ef, vf, n_vars, False, True, n_cons)

    n_groups = n_vars // n_vars_small_s
    ff = pl.pallas_call(
        _means_body,
        grid=(n_groups,),
        in_specs=[
            pl.BlockSpec((n_vars_small_s, EMB), lambda i: (i, 0)),
            pl.BlockSpec((n_cons_small_s, EMB), lambda i: (i, 0)),
        ],
        out_specs=pl.BlockSpec((1, 1, 2 * EMB), lambda i: (i, 0, 0)),
        out_shape=jax.ShapeDtypeStruct((n_groups, 1, 2 * EMB), jnp.float32),
    )(vf, cf).reshape(n_groups, 2 * EMB)

    out = pl.pallas_call(
        _readout_body,
        in_specs=[
            pl.BlockSpec((n_groups, 2 * EMB), lambda i: (0, 0)),
            pl.BlockSpec((2 * EMB, EMB), lambda i: (0, 0)),
            pl.BlockSpec((1, EMB), lambda i: (0, 0)),
            pl.BlockSpec((EMB, 1), lambda i: (0, 0)),
        ],
        grid=(1,),
        out_specs=pl.BlockSpec((n_groups, 1), lambda i: (0, 0)),
        out_shape=jax.ShapeDtypeStruct((n_groups, 1), jnp.float32),
    )(ff, p['fo1_W'], p['fo1_b'].reshape(1, EMB), p['fo2_W'])
    return out
